# baseline (speedup 1.0000x reference)
import functools

import jax
import jax.numpy as jnp
from jax import lax
from jax.experimental import pallas as pl
from jax.experimental.pallas import tpu as pltpu
from jax.experimental.pallas import tpu_sc as plsc

_NC = 2
_NS = 16
_L = 16
_NW = _NC * _NS

_N, _D = 200000, 64
_ROWS_W = _N // _NW
_RSUB = 125
_NSUB = _ROWS_W // _RSUB
_SUB = _RSUB * _D
_LB = _SUB + 16


def _base_tc_kernel(mask_ref, base_ref, carry):
    i = pl.program_id(0)

    @pl.when(i == 0)
    def _():
        carry[0] = 0

    base_ref[i] = carry[0]
    carry[0] = carry[0] + jnp.sum(mask_ref[...])


def _compute_bases(mask_i32_3d):
    return pl.pallas_call(
        _base_tc_kernel,
        grid=(_NW,),
        in_specs=[pl.BlockSpec((1, _ROWS_W, _D), lambda i: (i, 0, 0))],
        out_specs=pl.BlockSpec(memory_space=pltpu.SMEM),
        out_shape=jax.ShapeDtypeStruct((_NW,), jnp.int32),
        scratch_shapes=[pltpu.SMEM((1,), jnp.int32)],
        compiler_params=pltpu.CompilerParams(
            dimension_semantics=("arbitrary",),
        ),
    )(mask_i32_3d)


def _make_sc_body(cap):
    def _sc_body(data_hbm, mask_hbm, imps_hbm, base_hbm, out_hbm,
                 dbuf, mbuf, ibuf, obuf, bbuf):
        c = lax.axis_index("c")
        s = lax.axis_index("s")
        wid = s * _NC + c
        row_start = wid * _ROWS_W

        grp = (wid // _L) * _L
        pltpu.sync_copy(base_hbm.at[pl.ds(grp, _L)], bbuf)
        lane = jnp.full((_L,), wid - grp, dtype=jnp.int32)
        off0 = plsc.load_gather(bbuf, [lane])[0]

        def inner(g, st):
            for j in range(_D // _L):
                m = mbuf[g, pl.ds(j * _L, _L)]
                mb = m > 0
                cs = plsc.cumsum(m)
                idx = (st + cs) - m
                v = plsc.load_gather(ibuf, [idx], mask=mb)
                d = dbuf[g, pl.ds(j * _L, _L)]
                obuf[g, pl.ds(j * _L, _L)] = jnp.where(mb, v, d)
                st = st + cs[15]
            return st

        def outer(b, off):
            r0 = row_start + b * _RSUB
            pltpu.sync_copy(data_hbm.at[pl.ds(r0, _RSUB)], dbuf)
            pltpu.sync_copy(mask_hbm.at[pl.ds(r0, _RSUB)], mbuf)
            off_al = jnp.minimum((off // 8) * 8, cap)
            rem = off - off_al
            pltpu.sync_copy(imps_hbm.at[pl.ds(off_al, _LB)], ibuf)
            st = lax.fori_loop(0, _RSUB, inner, rem)
            pltpu.sync_copy(obuf, out_hbm.at[pl.ds(r0, _RSUB)])
            return off_al + st

        lax.fori_loop(0, _NSUB, outer, off0)

    return _sc_body


def kernel(data, mask, imps):
    mask_i32 = mask.astype(jnp.int32)
    bases = _compute_bases(mask_i32.reshape(_NW, _ROWS_W, _D))

    nnz = imps.shape[0]
    if nnz >= _LB:
        cap = ((nnz - _LB + 7) // 8) * 8
        imps_eff = imps
    else:
        cap = 0
        imps_eff = jnp.pad(imps, (0, _LB - nnz))

    mesh = plsc.VectorSubcoreMesh(
        core_axis_name="c", subcore_axis_name="s",
        num_cores=_NC, num_subcores=_NS,
    )
    sc = functools.partial(
        pl.kernel,
        mesh=mesh,
        out_type=jax.ShapeDtypeStruct((_N, _D), jnp.float32),
        scratch_types=[
            pltpu.VMEM((_RSUB, _D), jnp.float32),
            pltpu.VMEM((_RSUB, _D), jnp.int32),
            pltpu.VMEM((_LB,), jnp.float32),
            pltpu.VMEM((_RSUB, _D), jnp.float32),
            pltpu.VMEM((_L,), jnp.int32),
        ],
        compiler_params=pltpu.CompilerParams(
            needs_layout_passes=False, use_tc_tiling_on_sc=False),
    )(_make_sc_body(cap))
    return sc(data, mask_i32, imps_eff, bases)


# speedup vs baseline: 1.2330x; 1.2330x over previous
import functools

import jax
import jax.numpy as jnp
from jax import lax
from jax.experimental import pallas as pl
from jax.experimental.pallas import tpu as pltpu
from jax.experimental.pallas import tpu_sc as plsc

_NC = 2
_NS = 16
_L = 16
_NW = _NC * _NS

_N, _D = 200000, 64
_SB = 80
_SUBE = _SB * _D
_NSB = _N // _SB
_TCB = 20
_TCROWS = _SB * _TCB
_TCG = _NSB // _TCB
_NBASE = 2512
_W = _SUBE + 16
_QUOT, _REM = divmod(_NSB, _NW)


def _base_tc_kernel(mask_ref, base_ref, carry):
    i = pl.program_id(0)

    @pl.when(i == 0)
    def _():
        carry[0] = 0

    acc = carry[0]
    for t in range(_TCB):
        base_ref[i * _TCB + t] = acc
        acc = acc + jnp.sum(mask_ref[pl.ds(t * _SB, _SB), :])
    carry[0] = acc


def _compute_bases(mask_i32):
    return pl.pallas_call(
        _base_tc_kernel,
        grid=(_TCG,),
        in_specs=[pl.BlockSpec((_TCROWS, _D), lambda i: (i, 0))],
        out_specs=pl.BlockSpec(memory_space=pltpu.SMEM),
        out_shape=jax.ShapeDtypeStruct((_NBASE,), jnp.int32),
        scratch_shapes=[pltpu.SMEM((1,), jnp.int32)],
        compiler_params=pltpu.CompilerParams(
            dimension_semantics=("arbitrary",),
        ),
    )(mask_i32)


def _make_sc_body(cap):
    def _sc_body(data_hbm, mask_hbm, imps_hbm, base_hbm, out_hbm,
                 dbuf, mbuf, ibuf, obuf, bbuf):
        c = lax.axis_index("c")
        s = lax.axis_index("s")
        wid = s * _NC + c
        nblk = _QUOT + jnp.where(wid < _REM, 1, 0)
        sb0 = wid * _QUOT + jnp.minimum(wid, _REM)

        grp = (sb0 // _L) * _L
        pltpu.sync_copy(base_hbm.at[pl.ds(grp, _L)], bbuf)
        lane = jnp.broadcast_to(sb0 - grp, (_L,)).astype(jnp.int32)
        off0 = plsc.load_gather(bbuf, [lane])[0]

        def inner(g, st):
            for j in range(_D // _L):
                m = mbuf[g, pl.ds(j * _L, _L)]
                mb = m > 0
                cs = plsc.cumsum(m)
                idx = (st + cs) - m
                v = plsc.load_gather(ibuf, [idx], mask=mb)
                d = dbuf[g, pl.ds(j * _L, _L)]
                obuf[g, pl.ds(j * _L, _L)] = jnp.where(mb, v, d)
                st = st + cs[15]
            return st

        def outer(b, off):
            r0 = (sb0 + b) * _SB
            pltpu.sync_copy(data_hbm.at[pl.ds(r0, _SB)], dbuf)
            pltpu.sync_copy(mask_hbm.at[pl.ds(r0, _SB)], mbuf)
            off_al = jnp.minimum((off // 8) * 8, cap)
            rem = off - off_al
            pltpu.sync_copy(imps_hbm.at[pl.ds(off_al, _W)], ibuf)
            st = lax.fori_loop(0, _SB, inner, rem)
            pltpu.sync_copy(obuf, out_hbm.at[pl.ds(r0, _SB)])
            return off_al + st

        lax.fori_loop(0, nblk, outer, off0)

    return _sc_body


def kernel(data, mask, imps):
    mask_i32 = mask.astype(jnp.int32)
    bases = _compute_bases(mask_i32)

    nnz = imps.shape[0]
    if nnz >= _W:
        cap = ((nnz - _W + 7) // 8) * 8
        imps_eff = imps
    else:
        cap = 0
        imps_eff = jnp.pad(imps, (0, _W - nnz))

    mesh = plsc.VectorSubcoreMesh(
        core_axis_name="c", subcore_axis_name="s",
        num_cores=_NC, num_subcores=_NS,
    )
    sc = functools.partial(
        pl.kernel,
        mesh=mesh,
        out_type=jax.ShapeDtypeStruct((_N, _D), jnp.float32),
        scratch_types=[
            pltpu.VMEM((_SB, _D), jnp.float32),
            pltpu.VMEM((_SB, _D), jnp.int32),
            pltpu.VMEM((_W,), jnp.float32),
            pltpu.VMEM((_SB, _D), jnp.float32),
            pltpu.VMEM((_L,), jnp.int32),
        ],
        compiler_params=pltpu.CompilerParams(needs_layout_passes=False),
    )(_make_sc_body(cap))
    return sc(data, mask_i32, imps_eff, bases)


# speedup vs baseline: 2.0306x; 1.6468x over previous
import functools

import jax
import jax.numpy as jnp
from jax import lax
from jax.experimental import pallas as pl
from jax.experimental.pallas import tpu as pltpu
from jax.experimental.pallas import tpu_sc as plsc

_NC = 2
_NS = 16
_L = 16
_NW = _NC * _NS

_N, _D = 200000, 64
_BW = 128
_NB = _N // _BW
_TAIL = _N - _NB * _BW
_TAIL0 = _NB * _BW
_BLKE = _BW * _D
_WIN = _BLKE + 16
_QUOT, _REM = divmod(_NB, _NW)


def _worker_id():
    return lax.axis_index("s") * _NC + lax.axis_index("c")


def _count_body(mask_hbm, cnt_hbm, mbuf, cbuf):
    wid = _worker_id()
    nblk = _QUOT + jnp.where(wid < _REM, 1, 0)
    sb0 = wid * _QUOT + jnp.minimum(wid, _REM)

    def blk(b, acc):
        pltpu.sync_copy(mask_hbm.at[:, pl.ds((sb0 + b) * _BW, _BW)], mbuf)

        def col(j, a):
            for g in range(_BW // _L):
                a = a + mbuf[j, pl.ds(g * _L, _L)]
            return a

        return lax.fori_loop(0, _D, col, acc)

    acc = lax.fori_loop(0, nblk, blk, jnp.zeros((_L,), jnp.int32))
    cbuf[...] = acc
    pltpu.sync_copy(cbuf, cnt_hbm.at[pl.ds(wid * _L, _L)])


def _make_main_body(cap):
    def body(data_hbm, mask_hbm, imps_hbm, cnt_hbm, out_hbm,
             dbuf, mbuf, ibuf, obuf, cntb, dtb, mtb, otb):
        wid = _worker_id()
        nblk = _QUOT + jnp.where(wid < _REM, 1, 0)
        sb0 = wid * _QUOT + jnp.minimum(wid, _REM)

        pltpu.sync_copy(cnt_hbm, cntb)
        vec = jnp.zeros((_L,), jnp.int32)
        for w in range(_NW):
            vec = vec + jnp.where(w < wid, cntb[pl.ds(w * _L, _L)], 0)
        off0 = plsc.cumsum(vec)[15]

        def process_block(i0, off, width, db, mb, ob):
            ng = width // _L
            pltpu.sync_copy(data_hbm.at[:, pl.ds(i0, width)], db)
            pltpu.sync_copy(mask_hbm.at[:, pl.ds(i0, width)], mb)
            off_al = jnp.minimum((off // 8) * 8, cap)
            rem = off - off_al
            pltpu.sync_copy(imps_hbm.at[pl.ds(off_al, _WIN)], ibuf)

            def pcol(j, cs):
                return tuple(cs[g] + mb[j, pl.ds(g * _L, _L)]
                             for g in range(ng))

            zeros = jnp.zeros((_L,), jnp.int32)
            cnts = lax.fori_loop(0, _D, pcol, (zeros,) * ng)

            bases = []
            gb = rem
            for g in range(ng):
                iq = plsc.cumsum(cnts[g])
                bases.append(gb + (iq - cnts[g]))
                gb = gb + iq[15]
            total = gb - rem

            def mcol(j, runs):
                out = list(runs)
                for g in range(ng):
                    sl = pl.ds(g * _L, _L)
                    m = mb[j, sl]
                    mbool = m > 0
                    d = db[j, sl]
                    v = plsc.load_gather(ibuf, [out[g]], mask=mbool)
                    ob[j, sl] = jnp.where(mbool, v, d)
                    out[g] = out[g] + m
                return tuple(out)

            lax.fori_loop(0, _D, mcol, tuple(bases))
            pltpu.sync_copy(ob, out_hbm.at[:, pl.ds(i0, width)])
            return off_al + rem + total

        def blk(b, off):
            return process_block((sb0 + b) * _BW, off, _BW,
                                 dbuf, mbuf, obuf)

        off = lax.fori_loop(0, nblk, blk, off0)

        @pl.when(wid == _NW - 1)
        def _():
            process_block(_TAIL0, off, _TAIL, dtb, mtb, otb)

    return body


def kernel(data, mask, imps):
    data_t = data.T
    mask_t = mask.T.astype(jnp.int32)

    nnz = imps.shape[0]
    if nnz >= _WIN:
        cap = ((nnz - _WIN + 7) // 8) * 8
        imps_eff = imps
    else:
        cap = 0
        imps_eff = jnp.pad(imps, (0, _WIN - nnz))

    mesh = plsc.VectorSubcoreMesh(
        core_axis_name="c", subcore_axis_name="s",
        num_cores=_NC, num_subcores=_NS,
    )
    params = pltpu.CompilerParams(needs_layout_passes=False)

    counts = functools.partial(
        pl.kernel,
        mesh=mesh,
        out_type=jax.ShapeDtypeStruct((_NW * _L,), jnp.int32),
        scratch_types=[
            pltpu.VMEM((_D, _BW), jnp.int32),
            pltpu.VMEM((_L,), jnp.int32),
        ],
        compiler_params=params,
    )(_count_body)(mask_t)

    main = functools.partial(
        pl.kernel,
        mesh=mesh,
        out_type=jax.ShapeDtypeStruct((_D, _N), jnp.float32),
        scratch_types=[
            pltpu.VMEM((_D, _BW), jnp.float32),
            pltpu.VMEM((_D, _BW), jnp.int32),
            pltpu.VMEM((_WIN,), jnp.float32),
            pltpu.VMEM((_D, _BW), jnp.float32),
            pltpu.VMEM((_NW * _L,), jnp.int32),
            pltpu.VMEM((_D, _TAIL), jnp.float32),
            pltpu.VMEM((_D, _TAIL), jnp.int32),
            pltpu.VMEM((_D, _TAIL), jnp.float32),
        ],
        compiler_params=params,
    )(_make_main_body(cap))
    out_t = main(data_t, mask_t, imps_eff, counts)
    return out_t.T


# speedup vs baseline: 3.4468x; 1.6974x over previous
import functools

import jax
import jax.numpy as jnp
from jax import lax
from jax.experimental import pallas as pl
from jax.experimental.pallas import tpu as pltpu
from jax.experimental.pallas import tpu_sc as plsc

_NC = 2
_NS = 16
_L = 16
_NW = _NC * _NS

_N, _D = 200000, 64
_BW = 128
_NB = _N // _BW
_TAIL = _N - _NB * _BW
_TAIL0 = _NB * _BW
_BLKE = _BW * _D
_WIN = _BLKE + 16
_NG = _BW // _L
_BIG = (_NB - 48 * _NW) // 2


def _worker_id():
    return lax.axis_index("s") * _NC + lax.axis_index("c")


def _assignment(wid):
    nblk = 48 + 2 * jnp.where(wid < _BIG, 1, 0)
    sb0 = wid * 48 + 2 * jnp.minimum(wid, _BIG)
    return nblk, sb0


def _count_body(mask_hbm, cnt_hbm, mbuf0, mbuf1, cbuf, sem0, sem1):
    wid = _worker_id()
    nblk, sb0 = _assignment(wid)

    def msl(sb):
        return mask_hbm.at[:, pl.ds(sb * _BW, _BW)]

    pltpu.async_copy(msl(sb0), mbuf0, sem0)

    def count_one(sb, acc, mb, sem, nmb, nsem):
        pltpu.make_async_copy(msl(sb), mb, sem).wait()

        @pl.when(sb + 1 - sb0 < nblk)
        def _():
            pltpu.async_copy(msl(sb + 1), nmb, nsem)

        def col(j, a):
            for g in range(_NG):
                a = a + mb[j, pl.ds(g * _L, _L)]
            return a

        return lax.fori_loop(0, _D, col, acc)

    def pair(s, acc):
        sb = sb0 + 2 * s
        acc = count_one(sb, acc, mbuf0, sem0, mbuf1, sem1)
        acc = count_one(sb + 1, acc, mbuf1, sem1, mbuf0, sem0)
        return acc

    acc = lax.fori_loop(0, nblk // 2, pair, jnp.zeros((_L,), jnp.int32))
    cbuf[...] = acc
    pltpu.sync_copy(cbuf, cnt_hbm.at[pl.ds(wid * _L, _L)])


def _make_main_body(cap):
    def body(data_hbm, mask_hbm, imps_hbm, cnt_hbm, out_hbm,
             dbuf0, dbuf1, mbuf0, mbuf1, ibuf0, ibuf1, obuf0, obuf1,
             cntb, dtb, mtb, otb, sin0, sin1, sout0, sout1):
        wid = _worker_id()
        nblk, sb0 = _assignment(wid)

        pltpu.sync_copy(cnt_hbm, cntb)
        vec = jnp.zeros((_L,), jnp.int32)
        for w in range(_NW):
            vec = vec + jnp.where(w < wid, cntb[pl.ds(w * _L, _L)], 0)
        off0 = plsc.cumsum(vec)[15]
        al0 = pl.multiple_of(jnp.minimum((off0 // 8) * 8, cap), 8)

        def dsl(sb):
            return data_hbm.at[:, pl.ds(sb * _BW, _BW)]

        def msl(sb):
            return mask_hbm.at[:, pl.ds(sb * _BW, _BW)]

        def osl(sb):
            return out_hbm.at[:, pl.ds(sb * _BW, _BW)]

        def isl(al):
            return imps_hbm.at[pl.ds(al, _WIN)]

        pltpu.async_copy(dsl(sb0), dbuf0, sin0)
        pltpu.async_copy(msl(sb0), mbuf0, sin0)
        pltpu.async_copy(isl(al0), ibuf0, sin0)

        zeros = jnp.zeros((_L,), jnp.int32)

        def expand_block(db, mb, ib, ob, off, al, ng):
            def pcol(j, cs):
                return tuple(cs[g] + mb[j, pl.ds(g * _L, _L)]
                             for g in range(ng))

            cnts = lax.fori_loop(0, _D, pcol, (zeros,) * ng)
            rem = off - al
            bases = []
            gb = rem
            for g in range(ng):
                iq = plsc.cumsum(cnts[g])
                bases.append(gb + (iq - cnts[g]))
                gb = gb + iq[15]
            off2 = al + gb

            def mcol(j, runs):
                out = list(runs)
                for g in range(ng):
                    sl = pl.ds(g * _L, _L)
                    m = mb[j, sl]
                    mbool = m > 0
                    d = db[j, sl]
                    v = plsc.load_gather(ib, [out[g]], mask=mbool)
                    ob[j, sl] = jnp.where(mbool, v, d)
                    out[g] = out[g] + m
                return tuple(out)

            lax.fori_loop(0, _D, mcol, tuple(bases))
            return off2

        def do_block(b, off, al, db, mb, ib, ob, sin, sout,
                     ndb, nmb, nib, nsin):
            sb = sb0 + b
            pltpu.make_async_copy(dsl(sb), db, sin).wait()
            pltpu.make_async_copy(msl(sb), mb, sin).wait()
            pltpu.make_async_copy(isl(al), ib, sin).wait()

            @pl.when(b + 1 < nblk)
            def _():
                pltpu.async_copy(dsl(sb + 1), ndb, nsin)
                pltpu.async_copy(msl(sb + 1), nmb, nsin)

            def pcol(j, cs):
                return tuple(cs[g] + mb[j, pl.ds(g * _L, _L)]
                             for g in range(_NG))

            cnts = lax.fori_loop(0, _D, pcol, (zeros,) * _NG)
            rem = off - al
            bases = []
            gb = rem
            for g in range(_NG):
                iq = plsc.cumsum(cnts[g])
                bases.append(gb + (iq - cnts[g]))
                gb = gb + iq[15]
            off2 = al + gb
            al2 = pl.multiple_of(jnp.minimum((off2 // 8) * 8, cap), 8)

            @pl.when(b + 1 < nblk)
            def _():
                pltpu.async_copy(isl(al2), nib, nsin)

            @pl.when(b >= 2)
            def _():
                pltpu.make_async_copy(ob, osl(sb), sout).wait()

            def mcol(j, runs):
                out = list(runs)
                for g in range(_NG):
                    sl = pl.ds(g * _L, _L)
                    m = mb[j, sl]
                    mbool = m > 0
                    d = db[j, sl]
                    v = plsc.load_gather(ib, [out[g]], mask=mbool)
                    ob[j, sl] = jnp.where(mbool, v, d)
                    out[g] = out[g] + m
                return tuple(out)

            lax.fori_loop(0, _D, mcol, tuple(bases))
            pltpu.async_copy(ob, osl(sb), sout)
            return off2, al2

        def pair(s, carry):
            off, al = carry
            al = pl.multiple_of(al, 8)
            b = 2 * s
            off, al = do_block(b, off, al, dbuf0, mbuf0, ibuf0, obuf0,
                               sin0, sout0, dbuf1, mbuf1, ibuf1, sin1)
            al = pl.multiple_of(al, 8)
            off, al = do_block(b + 1, off, al, dbuf1, mbuf1, ibuf1, obuf1,
                               sin1, sout1, dbuf0, mbuf0, ibuf0, sin0)
            return (off, al)

        off_end, _al = lax.fori_loop(0, nblk // 2, pair, (off0, al0))

        pltpu.make_async_copy(obuf0, osl(sb0), sout0).wait()
        pltpu.make_async_copy(obuf1, osl(sb0), sout1).wait()

        @pl.when(wid == _NW - 1)
        def _():
            pltpu.sync_copy(data_hbm.at[:, pl.ds(_TAIL0, _TAIL)], dtb)
            pltpu.sync_copy(mask_hbm.at[:, pl.ds(_TAIL0, _TAIL)], mtb)
            al = pl.multiple_of(
                jnp.minimum((off_end // 8) * 8, cap), 8)
            pltpu.sync_copy(isl(al), ibuf0)
            expand_block(dtb, mtb, ibuf0, otb, off_end, al, _TAIL // _L)
            pltpu.sync_copy(otb, out_hbm.at[:, pl.ds(_TAIL0, _TAIL)])

    return body


def kernel(data, mask, imps):
    data_t = data.T
    mask_t = mask.T.astype(jnp.int32)

    nnz = imps.shape[0]
    if nnz >= _WIN:
        cap = ((nnz - _WIN + 7) // 8) * 8
        imps_eff = imps
    else:
        cap = 0
        imps_eff = jnp.pad(imps, (0, _WIN - nnz))

    mesh = plsc.VectorSubcoreMesh(
        core_axis_name="c", subcore_axis_name="s",
        num_cores=_NC, num_subcores=_NS,
    )
    params = pltpu.CompilerParams(needs_layout_passes=False)

    counts = functools.partial(
        pl.kernel,
        mesh=mesh,
        out_type=jax.ShapeDtypeStruct((_NW * _L,), jnp.int32),
        scratch_types=[
            pltpu.VMEM((_D, _BW), jnp.int32),
            pltpu.VMEM((_D, _BW), jnp.int32),
            pltpu.VMEM((_L,), jnp.int32),
            pltpu.SemaphoreType.DMA,
            pltpu.SemaphoreType.DMA,
        ],
        compiler_params=params,
    )(_count_body)(mask_t)

    main = functools.partial(
        pl.kernel,
        mesh=mesh,
        out_type=jax.ShapeDtypeStruct((_D, _N), jnp.float32),
        scratch_types=[
            pltpu.VMEM((_D, _BW), jnp.float32),
            pltpu.VMEM((_D, _BW), jnp.float32),
            pltpu.VMEM((_D, _BW), jnp.int32),
            pltpu.VMEM((_D, _BW), jnp.int32),
            pltpu.VMEM((_WIN,), jnp.float32),
            pltpu.VMEM((_WIN,), jnp.float32),
            pltpu.VMEM((_D, _BW), jnp.float32),
            pltpu.VMEM((_D, _BW), jnp.float32),
            pltpu.VMEM((_NW * _L,), jnp.int32),
            pltpu.VMEM((_D, _TAIL), jnp.float32),
            pltpu.VMEM((_D, _TAIL), jnp.int32),
            pltpu.VMEM((_D, _TAIL), jnp.float32),
            pltpu.SemaphoreType.DMA,
            pltpu.SemaphoreType.DMA,
            pltpu.SemaphoreType.DMA,
            pltpu.SemaphoreType.DMA,
        ],
        compiler_params=params,
    )(_make_main_body(cap))
    out_t = main(data_t, mask_t, imps_eff, counts)
    return out_t.T
